# Initial kernel scaffold; baseline (speedup 1.0000x reference)
import jax
import jax.numpy as jnp
from jax.experimental import pallas as pl


def kernel(reco, input_data0, cluster_label0):
    raise NotImplementedError("write your pallas kernel here")


# speedup vs baseline: 2.2122x; 2.2122x over previous
import functools

import jax
import jax.numpy as jnp
from jax import lax
from jax.experimental import pallas as pl
from jax.experimental.pallas import tpu as pltpu
from jax.experimental.pallas import tpu_sc as plsc

N = 1_600_000
B = 32
K = 64
NSEG = B * K
ROW = 2 * NSEG
NW = 32
ROWS_PER_W = N // NW
CHUNK = 2000
NCHUNK = ROWS_PER_W // CHUNK
VPC = CHUNK // 16

_mesh = plsc.VectorSubcoreMesh(core_axis_name="c", subcore_axis_name="s")


@functools.partial(
    pl.kernel,
    mesh=_mesh,
    out_type=jax.ShapeDtypeStruct((NW, ROW), jnp.float32),
    compiler_params=pltpu.CompilerParams(needs_layout_passes=False),
    scratch_types=[
        pltpu.VMEM((CHUNK,), jnp.float32),
        pltpu.VMEM((CHUNK * 5,), jnp.float32),
        pltpu.VMEM((CHUNK * 6,), jnp.int32),
        pltpu.VMEM((16 * ROW,), jnp.float32),
    ],
)
def _seg_reduce(reco_h, inp_h, cl_h, out_h, reco_v, inp_v, cl_v, acc):
    c = lax.axis_index("c")
    s = lax.axis_index("s")
    wid = c * 16 + s
    base = wid * ROWS_PER_W

    zeros = jnp.zeros((16,), jnp.float32)
    ones = jnp.ones((16,), jnp.float32)
    lane = lax.iota(jnp.int32, 16)
    lane_base = lane * ROW

    def zbody(i, carry):
        acc[pl.ds(pl.multiple_of(i * 16, 16), 16)] = zeros
        return carry

    lax.fori_loop(0, ROW, zbody, 0)

    def chunk_body(ch, carry):
        r0 = base + ch * CHUNK
        pltpu.sync_copy(reco_h.at[pl.ds(r0, CHUNK)], reco_v)
        pltpu.sync_copy(inp_h.at[pl.ds(r0 * 5, CHUNK * 5)], inp_v)
        pltpu.sync_copy(cl_h.at[pl.ds(r0 * 6, CHUNK * 6)], cl_v)

        def vbody(i, carry2):
            p = pl.multiple_of(i * 16, 16)
            lanes = lane + i * 16
            r = reco_v[pl.ds(p, 16)]
            t = plsc.load_gather(inp_v, [lanes * 5 + 4])
            bb = plsc.load_gather(cl_v, [lanes * 6 + 3])
            cc = plsc.load_gather(cl_v, [lanes * 6 + 4])
            seg = bb * K + cc
            d = r - t
            idx = lane_base + seg
            plsc.addupdate_scatter(acc, [idx], d * d)
            plsc.addupdate_scatter(acc, [idx + NSEG], ones)
            return carry2

        lax.fori_loop(0, VPC, vbody, 0)
        return carry

    lax.fori_loop(0, NCHUNK, chunk_body, 0)

    def rbody(j, carry):
        p = pl.multiple_of(j * 16, 16)
        def lbody(l, v):
            return v + acc[pl.ds(l * ROW + p, 16)]
        acc[pl.ds(p, 16)] = lax.fori_loop(1, 16, lbody, acc[pl.ds(p, 16)])
        return carry

    lax.fori_loop(0, ROW // 16, rbody, 0)

    pltpu.sync_copy(acc.at[pl.ds(0, ROW)], out_h.at[wid])


def _epilogue(p_ref, a_ref, o_ref):
    p = p_ref[...]
    s = jnp.sum(p[:, :NSEG], axis=0, keepdims=True)
    cnt = jnp.sum(p[:, NSEG:], axis=0, keepdims=True)
    pres = cnt > 0.0
    mse = jnp.where(pres, s / jnp.maximum(cnt, 1.0), 0.0)
    a = a_ref[...]
    bsum = jnp.dot(mse, a, preferred_element_type=jnp.float32,
                   precision=lax.Precision.HIGHEST)
    ncl = jnp.dot(pres.astype(jnp.float32), a,
                  preferred_element_type=jnp.float32,
                  precision=lax.Precision.HIGHEST)
    bl = bsum / jnp.maximum(ncl, 1.0)
    bp = ncl > 0.0
    loss = jnp.sum(jnp.where(bp, bl, 0.0)) / jnp.maximum(
        jnp.sum(bp.astype(jnp.float32)), 1.0)
    o_ref[...] = jnp.full((1, 1), loss, jnp.float32)


def kernel(reco, input_data0, cluster_label0):
    parts = _seg_reduce(
        reco.reshape(-1).astype(jnp.float32),
        input_data0.reshape(-1).astype(jnp.float32),
        cluster_label0.reshape(-1).astype(jnp.int32),
    )
    a = (jnp.arange(NSEG, dtype=jnp.int32)[:, None] // K
         == jnp.arange(B, dtype=jnp.int32)[None, :]).astype(jnp.float32)
    loss2d = pl.pallas_call(
        _epilogue,
        out_shape=jax.ShapeDtypeStruct((1, 1), jnp.float32),
    )(parts, a)
    return loss2d[0, 0]
